# baseline (speedup 1.0000x reference)
import numpy as np
import jax
import jax.numpy as jnp
from jax import lax
from jax.experimental import pallas as pl
from jax.experimental.pallas import tpu as pltpu

VOCAB = 100000
ROWS, COLS = 128, 4096
TIMESTEPS = 1000

BR, BC = 8, 512

_ROTS = ((13, 15, 26, 6), (17, 29, 16, 24))


def _np_threefry(k0, k1, x0, x1):
    with np.errstate(over="ignore"):
        k0, k1 = np.uint32(k0), np.uint32(k1)
        x0, x1 = np.uint32(x0), np.uint32(x1)
        ks = (k0, k1, np.uint32(k0 ^ k1 ^ np.uint32(0x1BD11BDA)))
        x0 = x0 + ks[0]
        x1 = x1 + ks[1]
        for i in range(5):
            for r in _ROTS[i % 2]:
                x0 = x0 + x1
                x1 = (x1 << np.uint32(r)) | (x1 >> np.uint32(32 - r))
                x1 = x1 ^ x0
            x0 = x0 + ks[(i + 1) % 3]
            x1 = x1 + ks[(i + 2) % 3] + np.uint32(i + 1)
        return x0, x1


def _np_split(k):
    a0, b0 = _np_threefry(k[0], k[1], 0, 0)
    a1, b1 = _np_threefry(k[0], k[1], 0, 1)
    return (a0, b0), (a1, b1)


_KN, _KB = _np_split((np.uint32(0), np.uint32(42)))
_K1, _K2 = _np_split(_KN)


def _u32(v):
    return np.uint32(v & 0xFFFFFFFF)


def _tf_bits(key, iota_u32, off):
    k0, k1 = int(key[0]), int(key[1])
    ks = (_u32(k0), _u32(k1), _u32(k0 ^ k1 ^ 0x1BD11BDA))
    c1 = np.array(k1, np.uint32).view(np.int32).item()
    c0 = np.array((k0 + k1) & 0xFFFFFFFF, np.uint32).view(np.int32).item()
    x1 = iota_u32 + (off + np.int32(c1)).astype(jnp.uint32)
    x0 = iota_u32 + (off + np.int32(c0)).astype(jnp.uint32)
    for i in range(5):
        for j, r in enumerate(_ROTS[i % 2]):
            if i or j:
                x0 = x0 + x1
            x1 = ((x1 << _u32(r)) | (x1 >> _u32(32 - r))) ^ x0
        x0 = x0 + ks[(i + 1) % 3]
        x1 = x1 + _u32(int(ks[(i + 2) % 3]) + i + 1)
    return x0 ^ x1


def _umod_vocab(bits_u32):
    h = (bits_u32 >> _u32(18)).astype(jnp.int32)
    l = (bits_u32 & _u32(0x3FFFF)).astype(jnp.int32)
    w = h * np.int32((1 << 18) % VOCAB) + l
    f = w.astype(jnp.float32)
    q = (f * np.float32((1.0 + 4e-5) / VOCAB)).astype(jnp.int32)
    r = w - q * np.int32(VOCAB)
    return r + ((r >> np.int32(31)) & np.int32(VOCAB))


GB = 16
RB = ROWS // GB


def _body(t_ref, ab_ref, x_ref, o_ref):
    base = pl.program_id(0) * np.int32(RB * COLS)
    t = t_ref[:]
    steps = lax.broadcasted_iota(jnp.int32, (RB, TIMESTEPS), 1)
    ab = ab_ref[:]
    a_row = jnp.sum(jnp.where(t == steps, ab, 0.0), axis=1, keepdims=True)
    thr = jnp.ceil(a_row * np.float32(1 << 23)).astype(jnp.uint32)
    thr_s = thr << _u32(9)

    iota = (lax.broadcasted_iota(jnp.int32, (BR, BC), 0) * COLS
            + lax.broadcasted_iota(jnp.int32, (BR, BC), 1)).astype(jnp.uint32)
    for r0 in range(0, RB, BR):
        thr_blk = thr_s[r0:r0 + BR, :]
        for c0 in range(0, COLS, BC):
            off = base + np.int32(r0 * COLS + c0)
            noise = _umod_vocab(_tf_bits(_K2, iota, off))
            ubits = _tf_bits(_KB, iota, off)
            keep = ubits < thr_blk
            o_ref[r0:r0 + BR, c0:c0 + BC] = jnp.where(
                keep, x_ref[r0:r0 + BR, c0:c0 + BC], noise)


@jax.jit
def kernel(x_start, t, alpha_bars):
    x_start = x_start.astype(jnp.int32)
    t2 = t.astype(jnp.int32).reshape(ROWS, 1)
    ab2 = alpha_bars.astype(jnp.float32).reshape(1, TIMESTEPS)
    return pl.pallas_call(
        _body,
        grid=(GB,),
        in_specs=[
            pl.BlockSpec((RB, 1), lambda i: (i, 0)),
            pl.BlockSpec((1, TIMESTEPS), lambda i: (0, 0)),
            pl.BlockSpec((RB, COLS), lambda i: (i, 0)),
        ],
        out_specs=pl.BlockSpec((RB, COLS), lambda i: (i, 0)),
        out_shape=jax.ShapeDtypeStruct((ROWS, COLS), jnp.int32),
    )(t2, ab2, x_start)


# speedup vs baseline: 1.0171x; 1.0171x over previous
import numpy as np
import jax
import jax.numpy as jnp
from jax import lax
from jax.experimental import pallas as pl
from jax.experimental.pallas import tpu as pltpu

VOCAB = 100000
ROWS, COLS = 128, 4096
TIMESTEPS = 1000

BR, BC = 8, 512

_ROTS = ((13, 15, 26, 6), (17, 29, 16, 24))


def _np_threefry(k0, k1, x0, x1):
    with np.errstate(over="ignore"):
        k0, k1 = np.uint32(k0), np.uint32(k1)
        x0, x1 = np.uint32(x0), np.uint32(x1)
        ks = (k0, k1, np.uint32(k0 ^ k1 ^ np.uint32(0x1BD11BDA)))
        x0 = x0 + ks[0]
        x1 = x1 + ks[1]
        for i in range(5):
            for r in _ROTS[i % 2]:
                x0 = x0 + x1
                x1 = (x1 << np.uint32(r)) | (x1 >> np.uint32(32 - r))
                x1 = x1 ^ x0
            x0 = x0 + ks[(i + 1) % 3]
            x1 = x1 + ks[(i + 2) % 3] + np.uint32(i + 1)
        return x0, x1


def _np_split(k):
    a0, b0 = _np_threefry(k[0], k[1], 0, 0)
    a1, b1 = _np_threefry(k[0], k[1], 0, 1)
    return (a0, b0), (a1, b1)


_KN, _KB = _np_split((np.uint32(0), np.uint32(42)))
_K1, _K2 = _np_split(_KN)


def _u32(v):
    return np.uint32(v & 0xFFFFFFFF)


def _tf_bits(key, iota_u32, off):
    k0, k1 = int(key[0]), int(key[1])
    ks = (_u32(k0), _u32(k1), _u32(k0 ^ k1 ^ 0x1BD11BDA))
    c1 = np.array(k1, np.uint32).view(np.int32).item()
    c0 = np.array((k0 + k1) & 0xFFFFFFFF, np.uint32).view(np.int32).item()
    x1 = iota_u32 + (off + np.int32(c1)).astype(jnp.uint32)
    x0 = iota_u32 + (off + np.int32(c0)).astype(jnp.uint32)
    for i in range(5):
        for j, r in enumerate(_ROTS[i % 2]):
            if i or j:
                x0 = x0 + x1
            x1 = ((x1 << _u32(r)) | (x1 >> _u32(32 - r))) ^ x0
        x0 = x0 + ks[(i + 1) % 3]
        x1 = x1 + _u32(int(ks[(i + 2) % 3]) + i + 1)
    return x0 ^ x1


def _umod_vocab(bits_u32):
    h = (bits_u32 >> _u32(18)).astype(jnp.int32)
    l = (bits_u32 & _u32(0x3FFFF)).astype(jnp.int32)
    w = h * np.int32((1 << 18) % VOCAB) + l
    f = w.astype(jnp.float32)
    q = (f * np.float32((1.0 + 4e-5) / VOCAB)).astype(jnp.int32)
    r = w - q * np.int32(VOCAB)
    return r + ((r >> np.int32(31)) & np.int32(VOCAB))


GB = 8
RB = ROWS // GB


def _body(t_ref, ab_ref, x_ref, o_ref):
    base = pl.program_id(0) * np.int32(RB * COLS)
    t = t_ref[:]
    steps = lax.broadcasted_iota(jnp.int32, (RB, TIMESTEPS), 1)
    ab = ab_ref[:]
    a_row = jnp.sum(jnp.where(t == steps, ab, 0.0), axis=1, keepdims=True)
    thr = jnp.ceil(a_row * np.float32(1 << 23)).astype(jnp.uint32)
    thr_s = thr << _u32(9)

    iota = (lax.broadcasted_iota(jnp.int32, (BR, BC), 0) * COLS
            + lax.broadcasted_iota(jnp.int32, (BR, BC), 1)).astype(jnp.uint32)
    for r0 in range(0, RB, BR):
        thr_blk = thr_s[r0:r0 + BR, :]
        for c0 in range(0, COLS, BC):
            off = base + np.int32(r0 * COLS + c0)
            noise = _umod_vocab(_tf_bits(_K2, iota, off))
            ubits = _tf_bits(_KB, iota, off)
            keep = ubits < thr_blk
            o_ref[r0:r0 + BR, c0:c0 + BC] = jnp.where(
                keep, x_ref[r0:r0 + BR, c0:c0 + BC], noise)


@jax.jit
def kernel(x_start, t, alpha_bars):
    x_start = x_start.astype(jnp.int32)
    t2 = t.astype(jnp.int32).reshape(ROWS, 1)
    ab2 = alpha_bars.astype(jnp.float32).reshape(1, TIMESTEPS)
    return pl.pallas_call(
        _body,
        grid=(GB,),
        in_specs=[
            pl.BlockSpec((RB, 1), lambda i: (i, 0)),
            pl.BlockSpec((1, TIMESTEPS), lambda i: (0, 0)),
            pl.BlockSpec((RB, COLS), lambda i: (i, 0)),
        ],
        out_specs=pl.BlockSpec((RB, COLS), lambda i: (i, 0)),
        out_shape=jax.ShapeDtypeStruct((ROWS, COLS), jnp.int32),
    )(t2, ab2, x_start)
